# baseline (speedup 1.0000x reference)
import functools

import jax
import jax.numpy as jnp
import numpy as np
from jax import lax
from jax.experimental import pallas as pl
from jax.experimental.pallas import tpu as pltpu
from jax.experimental.pallas import tpu_sc as plsc

ROWS, COLS = 2048, 8192
TOTAL = ROWS * COLS
NC, NS, L = 2, 16, 16
NW = NC * NS
PER_W = TOTAL // NW
CHUNK = 8192
NCHUNK = PER_W // CHUNK
NBINS = 65536
HR, HC = 512, 128

SIG_BIAS = 5.0
_S_POS = float(1.0 / (1.0 + np.exp(np.float64(-5.0))))
_S_NEG = float(1.0 / (1.0 + np.exp(np.float64(5.0))))

_MESH = plsc.VectorSubcoreMesh(core_axis_name="c", subcore_axis_name="s")


def _hist_prologue(sid, hist, rowidx, shared):
    zeros16 = jnp.zeros((L,), jnp.int32)
    iota16 = lax.iota(jnp.int32, L)

    @plsc.parallel_loop(0, NBINS // L, unroll=8)
    def _zero(i):
        hist[i >> 3, pl.ds((i & 7) * L, L)] = zeros16

    @plsc.parallel_loop(0, HR // L, unroll=4)
    def _ridx(i):
        rowidx[i >> 3, pl.ds((i & 7) * L, L)] = iota16 + i * L

    @pl.when(sid == 0)
    def _():
        pltpu.sync_copy(hist, shared)

    plsc.subcore_barrier()


def _hist_epilogue(sid, cid, hist, rowidx, shared, out_hbm):
    for i in range(HR // HC):
        pltpu.sync_copy(hist.at[pl.ds(i * HC, HC)], shared.at[rowidx.at[i]],
                        add=True)
    plsc.subcore_barrier()

    @pl.when(sid == 0)
    def _():
        pltpu.sync_copy(shared, out_hbm.at[cid])


def _sc_pass1_body(pre_hbm, fin_hbm, hist_out, bits_out, pbuf, fbuf, bbuf,
                   hist, rowidx, shared, sp0, sp1, sf0, sf1, sb0, sb1):
    cid = lax.axis_index("c")
    sid = lax.axis_index("s")
    wid = sid * NC + cid
    row0 = wid * (ROWS // NW)
    ones16 = jnp.ones((L,), jnp.int32)

    _hist_prologue(sid, hist, rowidx, shared)

    sems_p = (sp0, sp1)
    sems_f = (sf0, sf1)
    sems_b = (sb0, sb1)

    def _in_copies(j, slot):
        row = row0 + j
        cp = pltpu.make_async_copy(pre_hbm.at[pl.ds(row, 1), :],
                                   pbuf.at[slot], sems_p[slot])
        cf = pltpu.make_async_copy(fin_hbm.at[pl.ds(row, 1), :],
                                   fbuf.at[slot], sems_f[slot])
        return cp, cf

    def _out_copy(j, slot):
        return pltpu.make_async_copy(bbuf.at[slot], bits_out.at[wid, j],
                                     sems_b[slot])

    for b in range(2):
        cp, cf = _in_copies(jnp.int32(b), b)
        cp.start()
        cf.start()

    def _compute(slot):
        @plsc.parallel_loop(0, CHUNK // L, unroll=16)
        def inner(v):
            p = pbuf[slot, 0, pl.ds(v * L, L)]
            f = fbuf[slot, 0, pl.ds(v * L, L)]
            bits = lax.bitcast_convert_type(jnp.abs(f - p), jnp.int32)
            bbuf[slot, pl.ds(v * L, L)] = bits
            hi = lax.shift_right_logical(bits, jnp.int32(16))
            plsc.addupdate_scatter(
                hist, [lax.shift_right_logical(hi, jnp.int32(7)),
                       lax.bitwise_and(hi, jnp.int32(HC - 1))],
                ones16)

    def outer(i, carry):
        for b in range(2):
            j = i * 2 + b
            cp, cf = _in_copies(j, b)
            cp.wait()
            cf.wait()

            @pl.when(j >= 2)
            def _():
                _out_copy(j - 2, b).wait()

            _compute(b)
            _out_copy(j, b).start()

            @pl.when(j + 2 < NCHUNK)
            def _():
                cp2, cf2 = _in_copies(j + 2, b)
                cp2.start()
                cf2.start()

        return carry

    lax.fori_loop(0, NCHUNK // 2, outer, 0)

    for b in range(2):
        _out_copy(jnp.int32(NCHUNK - 2 + b), b).wait()

    _hist_epilogue(sid, cid, hist, rowidx, shared, hist_out)


def _sc_pass2_body(bits_hbm, bsel_hbm, hist_out, bbuf, hist, bvbuf,
                   rowidx, shared, sb0, sb1):
    cid = lax.axis_index("c")
    sid = lax.axis_index("s")
    wid = sid * NC + cid
    ones16 = jnp.ones((L,), jnp.int32)

    _hist_prologue(sid, hist, rowidx, shared)

    pltpu.sync_copy(bsel_hbm, bvbuf)
    bv = bvbuf[...]

    sems = (sb0, sb1)

    def _in_copy(j, slot):
        return pltpu.make_async_copy(bits_hbm.at[wid, j], bbuf.at[slot],
                                     sems[slot])

    for b in range(2):
        _in_copy(jnp.int32(b), b).start()

    def _compute(slot):
        @plsc.parallel_loop(0, CHUNK // L, unroll=16)
        def inner(v):
            bits = bbuf[slot, pl.ds(v * L, L)]
            hi = lax.shift_right_logical(bits, jnp.int32(16))
            lo = lax.bitwise_and(bits, jnp.int32(0xFFFF))
            plsc.addupdate_scatter(
                hist, [lax.shift_right_logical(lo, jnp.int32(7)),
                       lax.bitwise_and(lo, jnp.int32(HC - 1))],
                ones16, mask=hi == bv)

    def outer(i, carry):
        for b in range(2):
            j = i * 2 + b
            _in_copy(j, b).wait()
            _compute(b)

            @pl.when(j + 2 < NCHUNK)
            def _():
                _in_copy(j + 2, b).start()

        return carry

    lax.fori_loop(0, NCHUNK // 2, outer, 0)

    _hist_epilogue(sid, cid, hist, rowidx, shared, hist_out)


_hist_pass1 = pl.kernel(
    _sc_pass1_body,
    out_type=[jax.ShapeDtypeStruct((NC, HR, HC), jnp.int32),
              jax.ShapeDtypeStruct((NW, NCHUNK, CHUNK), jnp.int32)],
    mesh=_MESH,
    scratch_types=[
        pltpu.VMEM((2, 1, CHUNK), jnp.float32),
        pltpu.VMEM((2, 1, CHUNK), jnp.float32),
        pltpu.VMEM((2, CHUNK), jnp.int32),
        pltpu.VMEM((HR, HC), jnp.int32),
        pltpu.VMEM((HR // HC, HC), jnp.int32),
        pltpu.VMEM_SHARED((HR, HC), jnp.int32),
        pltpu.SemaphoreType.DMA,
        pltpu.SemaphoreType.DMA,
        pltpu.SemaphoreType.DMA,
        pltpu.SemaphoreType.DMA,
        pltpu.SemaphoreType.DMA,
        pltpu.SemaphoreType.DMA,
    ],
    compiler_params=pltpu.CompilerParams(needs_layout_passes=False),
)

_hist_pass2 = pl.kernel(
    _sc_pass2_body,
    out_type=jax.ShapeDtypeStruct((NC, HR, HC), jnp.int32),
    mesh=_MESH,
    scratch_types=[
        pltpu.VMEM((2, CHUNK), jnp.int32),
        pltpu.VMEM((HR, HC), jnp.int32),
        pltpu.VMEM((L,), jnp.int32),
        pltpu.VMEM((HR // HC, HC), jnp.int32),
        pltpu.VMEM_SHARED((HR, HC), jnp.int32),
        pltpu.SemaphoreType.DMA,
        pltpu.SemaphoreType.DMA,
    ],
    compiler_params=pltpu.CompilerParams(needs_layout_passes=False),
)


def _suffix_select(h, kk):
    m_col = (lax.broadcasted_iota(jnp.int32, (HC, HC), 0)
             > lax.broadcasted_iota(jnp.int32, (HC, HC), 1)).astype(jnp.float32)
    s_in = jnp.dot(h, m_col, precision=lax.Precision.HIGHEST)
    rowsum = jnp.sum(h, axis=1, keepdims=True)
    m_row = (lax.broadcasted_iota(jnp.int32, (HR, HR), 1)
             > lax.broadcasted_iota(jnp.int32, (HR, HR), 0)).astype(jnp.float32)
    row_suf = jnp.dot(m_row, rowsum, precision=lax.Precision.HIGHEST)
    s_tot = row_suf + s_in
    kf = kk.astype(jnp.float32)
    cond = jnp.logical_and(s_tot < kf, s_tot + h >= kf)
    bidx = (lax.broadcasted_iota(jnp.int32, (HR, HC), 0) * HC
            + lax.broadcasted_iota(jnp.int32, (HR, HC), 1))
    bsel = jnp.sum(jnp.where(cond, bidx, 0))
    nabove = jnp.sum(jnp.where(cond, s_tot, 0.0))
    return bsel, nabove


def _merge_hists(h_ref):
    def body(i, acc):
        return acc + h_ref[i].astype(jnp.float32)

    return lax.fori_loop(0, NC, body, jnp.zeros((HR, HC), jnp.float32))


def _select1_body(h_ref, k_ref, bvec_ref, nab_ref):
    h = _merge_hists(h_ref)
    kk = k_ref[0, 0]
    bsel, nabove = _suffix_select(h, kk)
    bvec_ref[...] = jnp.full((1, HC), bsel, jnp.int32)
    nab_ref[0, 0] = nabove.astype(jnp.int32)


def _select2_compute(h_ref, k_ref, bsel_ref, nab_ref):
    h = _merge_hists(h_ref)
    kk2 = k_ref[0, 0] - nab_ref[0, 0]
    lsel, nabove2 = _suffix_select(h, kk2)
    t_bits = lax.shift_left(bsel_ref[0, 0], jnp.int32(16)) | lsel
    thr = lax.bitcast_convert_type(t_bits, jnp.float32)
    c_above = nab_ref[0, 0].astype(jnp.float32) + nabove2
    return thr, c_above / jnp.float32(TOTAL)


def _select1(hists, kk):
    return pl.pallas_call(
        _select1_body,
        out_shape=[jax.ShapeDtypeStruct((1, HC), jnp.int32),
                   jax.ShapeDtypeStruct((1, 1), jnp.int32)],
        in_specs=[pl.BlockSpec(memory_space=pltpu.VMEM),
                  pl.BlockSpec(memory_space=pltpu.SMEM)],
        out_specs=[pl.BlockSpec(memory_space=pltpu.VMEM),
                   pl.BlockSpec(memory_space=pltpu.SMEM)],
    )(hists, kk)


_BLK_R = 128


def _elem_body(h_ref, k_ref, bsel_ref, nab_ref, pre_ref, fin_ref,
               interp_ref, logit_ref, srnd_ref, prop_ref, thr_smem):
    @pl.when(pl.program_id(0) == 0)
    def _():
        thr, prop = _select2_compute(h_ref, k_ref, bsel_ref, nab_ref)
        thr_smem[0] = thr
        prop_ref[0, 0] = prop

    t = thr_smem[0]
    p = pre_ref[...]
    tv = fin_ref[...] - p
    m = jnp.abs(tv) > t
    interp_ref[...] = p + tv * jnp.where(m, jnp.float32(_S_POS),
                                         jnp.float32(_S_NEG))
    logit_ref[...] = jnp.where(m, jnp.float32(SIG_BIAS), jnp.float32(-SIG_BIAS))
    srnd_ref[...] = jnp.where(m, jnp.float32(1.0), jnp.float32(0.0))


def _elementwise(hists2, kk, bvec, nab, pretrained, finetuned):
    grid = (ROWS // _BLK_R,)
    blk = pl.BlockSpec((_BLK_R, COLS), lambda i: (i, 0))
    const = lambda i: (0, 0)
    out = pl.pallas_call(
        _elem_body,
        grid=grid,
        out_shape=[jax.ShapeDtypeStruct((ROWS, COLS), jnp.float32)] * 3
        + [jax.ShapeDtypeStruct((1, 1), jnp.float32)],
        in_specs=[pl.BlockSpec((NC, HR, HC), lambda i: (0, 0, 0)),
                  pl.BlockSpec(memory_space=pltpu.SMEM),
                  pl.BlockSpec(memory_space=pltpu.SMEM),
                  pl.BlockSpec(memory_space=pltpu.SMEM),
                  blk, blk],
        out_specs=[blk, blk, blk,
                   pl.BlockSpec((1, 1), const, memory_space=pltpu.SMEM)],
        scratch_shapes=[pltpu.SMEM((1,), jnp.float32)],
    )(hists2, kk, bvec, nab, pretrained, finetuned)
    return out


def kernel(pretrained, finetuned, k):
    kk = jnp.asarray(k, jnp.int32).reshape(1, 1)

    h1, bits = _hist_pass1(pretrained, finetuned)
    bvec_row, nab1 = _select1(h1, kk)
    bvec = bvec_row[0, :L]

    h2 = _hist_pass2(bits, bvec)
    interp, logits, srnd, prop = _elementwise(h2, kk, bvec_row[:, :1], nab1,
                                              pretrained, finetuned)
    return interp, logits, srnd, prop.reshape(())


# speedup vs baseline: 1.0142x; 1.0142x over previous
import functools

import jax
import jax.numpy as jnp
import numpy as np
from jax import lax
from jax.experimental import pallas as pl
from jax.experimental.pallas import tpu as pltpu
from jax.experimental.pallas import tpu_sc as plsc

ROWS, COLS = 2048, 8192
TOTAL = ROWS * COLS
NC, NS, L = 2, 16, 16
NW = NC * NS
PER_W = TOTAL // NW
CHUNK = 8192
NCHUNK = PER_W // CHUNK
NBINS = 65536
HR, HC = 512, 128

SIG_BIAS = 5.0
_S_POS = float(1.0 / (1.0 + np.exp(np.float64(-5.0))))
_S_NEG = float(1.0 / (1.0 + np.exp(np.float64(5.0))))

_MESH = plsc.VectorSubcoreMesh(core_axis_name="c", subcore_axis_name="s")


def _hist_prologue(sid, hist, rowidx, shared):
    zeros16 = jnp.zeros((L,), jnp.int32)
    iota16 = lax.iota(jnp.int32, L)

    @plsc.parallel_loop(0, NBINS // L, unroll=8)
    def _zero(i):
        hist[i >> 3, pl.ds((i & 7) * L, L)] = zeros16

    @plsc.parallel_loop(0, HR // L, unroll=4)
    def _ridx(i):
        rowidx[i >> 3, pl.ds((i & 7) * L, L)] = iota16 + i * L

    @pl.when(sid == 0)
    def _():
        pltpu.sync_copy(hist, shared)

    plsc.subcore_barrier()


def _hist_epilogue(sid, cid, hist, rowidx, shared, out_hbm):
    for i in range(HR // HC):
        pltpu.sync_copy(hist.at[pl.ds(i * HC, HC)], shared.at[rowidx.at[i]],
                        add=True)
    plsc.subcore_barrier()

    @pl.when(sid == 0)
    def _():
        pltpu.sync_copy(shared, out_hbm.at[cid])


def _sc_hist_body(use_mask, pre_hbm, fin_hbm, bsel_hbm, out_hbm,
                  pbuf, fbuf, hist, bvbuf, rowidx, shared, sp0, sp1, sf0, sf1):
    cid = lax.axis_index("c")
    sid = lax.axis_index("s")
    wid = sid * NC + cid
    row0 = wid * (ROWS // NW)
    ones16 = jnp.ones((L,), jnp.int32)

    _hist_prologue(sid, hist, rowidx, shared)

    pltpu.sync_copy(bsel_hbm, bvbuf)
    bv = bvbuf[...]

    sems_p = (sp0, sp1)
    sems_f = (sf0, sf1)

    def _copies(j, slot):
        row = row0 + j
        cp = pltpu.make_async_copy(pre_hbm.at[pl.ds(row, 1), :],
                                   pbuf.at[slot], sems_p[slot])
        cf = pltpu.make_async_copy(fin_hbm.at[pl.ds(row, 1), :],
                                   fbuf.at[slot], sems_f[slot])
        return cp, cf

    for b in range(2):
        cp, cf = _copies(jnp.int32(b), b)
        cp.start()
        cf.start()

    def _compute(slot):
        @plsc.parallel_loop(0, CHUNK // L, unroll=16)
        def inner(v):
            p = pbuf[slot, 0, pl.ds(v * L, L)]
            f = fbuf[slot, 0, pl.ds(v * L, L)]
            bits = lax.bitcast_convert_type(jnp.abs(f - p), jnp.int32)
            hi = lax.shift_right_logical(bits, jnp.int32(16))
            if use_mask:
                lo = lax.bitwise_and(bits, jnp.int32(0xFFFF))
                plsc.addupdate_scatter(
                    hist, [lax.shift_right_logical(lo, jnp.int32(7)),
                           lax.bitwise_and(lo, jnp.int32(HC - 1))],
                    ones16, mask=hi == bv)
            else:
                plsc.addupdate_scatter(
                    hist, [lax.shift_right_logical(hi, jnp.int32(7)),
                           lax.bitwise_and(hi, jnp.int32(HC - 1))],
                    ones16)

    def outer(i, carry):
        for b in range(2):
            j = i * 2 + b
            cp, cf = _copies(j, b)
            cp.wait()
            cf.wait()
            _compute(b)

            @pl.when(j + 2 < NCHUNK)
            def _():
                cp2, cf2 = _copies(j + 2, b)
                cp2.start()
                cf2.start()

        return carry

    lax.fori_loop(0, NCHUNK // 2, outer, 0)

    _hist_epilogue(sid, cid, hist, rowidx, shared, out_hbm)


def _make_sc_hist(use_mask):
    return functools.partial(
        pl.kernel,
        functools.partial(_sc_hist_body, use_mask),
        out_type=jax.ShapeDtypeStruct((NC, HR, HC), jnp.int32),
        mesh=_MESH,
        scratch_types=[
            pltpu.VMEM((2, 1, CHUNK), jnp.float32),
            pltpu.VMEM((2, 1, CHUNK), jnp.float32),
            pltpu.VMEM((HR, HC), jnp.int32),
            pltpu.VMEM((L,), jnp.int32),
            pltpu.VMEM((HR // HC, HC), jnp.int32),
            pltpu.VMEM_SHARED((HR, HC), jnp.int32),
            pltpu.SemaphoreType.DMA,
            pltpu.SemaphoreType.DMA,
            pltpu.SemaphoreType.DMA,
            pltpu.SemaphoreType.DMA,
        ],
        compiler_params=pltpu.CompilerParams(needs_layout_passes=False),
    )()


_hist_pass1 = _make_sc_hist(use_mask=False)
_hist_pass2 = _make_sc_hist(use_mask=True)


def _suffix_select(h, kk):
    m_col = (lax.broadcasted_iota(jnp.int32, (HC, HC), 0)
             > lax.broadcasted_iota(jnp.int32, (HC, HC), 1)).astype(jnp.float32)
    s_in = jnp.dot(h, m_col, precision=lax.Precision.HIGHEST)
    rowsum = jnp.sum(h, axis=1, keepdims=True)
    m_row = (lax.broadcasted_iota(jnp.int32, (HR, HR), 1)
             > lax.broadcasted_iota(jnp.int32, (HR, HR), 0)).astype(jnp.float32)
    row_suf = jnp.dot(m_row, rowsum, precision=lax.Precision.HIGHEST)
    s_tot = row_suf + s_in
    kf = kk.astype(jnp.float32)
    cond = jnp.logical_and(s_tot < kf, s_tot + h >= kf)
    bidx = (lax.broadcasted_iota(jnp.int32, (HR, HC), 0) * HC
            + lax.broadcasted_iota(jnp.int32, (HR, HC), 1))
    bsel = jnp.sum(jnp.where(cond, bidx, 0))
    nabove = jnp.sum(jnp.where(cond, s_tot, 0.0))
    return bsel, nabove


def _merge_hists(h_ref):
    def body(i, acc):
        return acc + h_ref[i].astype(jnp.float32)

    return lax.fori_loop(0, NC, body, jnp.zeros((HR, HC), jnp.float32))


def _select1_body(h_ref, k_ref, bvec_ref, nab_ref):
    h = _merge_hists(h_ref)
    kk = k_ref[0, 0]
    bsel, nabove = _suffix_select(h, kk)
    bvec_ref[...] = jnp.full((1, HC), bsel, jnp.int32)
    nab_ref[0, 0] = nabove.astype(jnp.int32)


def _select2_compute(h_ref, k_ref, bsel_ref, nab_ref):
    h = _merge_hists(h_ref)
    kk2 = k_ref[0, 0] - nab_ref[0, 0]
    lsel, nabove2 = _suffix_select(h, kk2)
    t_bits = lax.shift_left(bsel_ref[0, 0], jnp.int32(16)) | lsel
    thr = lax.bitcast_convert_type(t_bits, jnp.float32)
    c_above = nab_ref[0, 0].astype(jnp.float32) + nabove2
    return thr, c_above / jnp.float32(TOTAL)


def _select1(hists, kk):
    return pl.pallas_call(
        _select1_body,
        out_shape=[jax.ShapeDtypeStruct((1, HC), jnp.int32),
                   jax.ShapeDtypeStruct((1, 1), jnp.int32)],
        in_specs=[pl.BlockSpec(memory_space=pltpu.VMEM),
                  pl.BlockSpec(memory_space=pltpu.SMEM)],
        out_specs=[pl.BlockSpec(memory_space=pltpu.VMEM),
                   pl.BlockSpec(memory_space=pltpu.SMEM)],
    )(hists, kk)


_BLK_R = 128


def _elem_body(h_ref, k_ref, bsel_ref, nab_ref, pre_ref, fin_ref,
               interp_ref, logit_ref, srnd_ref, prop_ref, thr_smem):
    @pl.when(pl.program_id(0) == 0)
    def _():
        thr, prop = _select2_compute(h_ref, k_ref, bsel_ref, nab_ref)
        thr_smem[0] = thr
        prop_ref[0, 0] = prop

    t = thr_smem[0]
    p = pre_ref[...]
    tv = fin_ref[...] - p
    m = jnp.abs(tv) > t
    interp_ref[...] = p + tv * jnp.where(m, jnp.float32(_S_POS),
                                         jnp.float32(_S_NEG))
    logit_ref[...] = jnp.where(m, jnp.float32(SIG_BIAS), jnp.float32(-SIG_BIAS))
    srnd_ref[...] = jnp.where(m, jnp.float32(1.0), jnp.float32(0.0))


def _elementwise(hists2, kk, bvec, nab, pretrained, finetuned):
    grid = (ROWS // _BLK_R,)
    blk = pl.BlockSpec((_BLK_R, COLS), lambda i: (i, 0))
    const = lambda i: (0, 0)
    out = pl.pallas_call(
        _elem_body,
        grid=grid,
        out_shape=[jax.ShapeDtypeStruct((ROWS, COLS), jnp.float32)] * 3
        + [jax.ShapeDtypeStruct((1, 1), jnp.float32)],
        in_specs=[pl.BlockSpec((NC, HR, HC), lambda i: (0, 0, 0)),
                  pl.BlockSpec(memory_space=pltpu.SMEM),
                  pl.BlockSpec(memory_space=pltpu.SMEM),
                  pl.BlockSpec(memory_space=pltpu.SMEM),
                  blk, blk],
        out_specs=[blk, blk, blk,
                   pl.BlockSpec((1, 1), const, memory_space=pltpu.SMEM)],
        scratch_shapes=[pltpu.SMEM((1,), jnp.float32)],
    )(hists2, kk, bvec, nab, pretrained, finetuned)
    return out


def kernel(pretrained, finetuned, k):
    kk = jnp.asarray(k, jnp.int32).reshape(1, 1)
    zero16 = jnp.full((L,), -1, jnp.int32)

    h1 = _hist_pass1(pretrained, finetuned, zero16)
    bvec_row, nab1 = _select1(h1, kk)
    bvec = bvec_row[0, :L]

    h2 = _hist_pass2(pretrained, finetuned, bvec)
    interp, logits, srnd, prop = _elementwise(h2, kk, bvec_row[:, :1], nab1,
                                              pretrained, finetuned)
    return interp, logits, srnd, prop.reshape(())
